# baseline (device time: 382157 ns/iter reference)
import jax
import jax.numpy as jnp
from jax import lax
from jax.experimental import pallas as pl
from jax.experimental.pallas import tpu as pltpu

N_DEV = 16
B_PER = 256
D = 256
H_PER = 512
BF16 = jnp.bfloat16
F32 = jnp.float32


def kernel(x, Win0, Wout0, Win1, Wout1, Win2, Wout2):
    def body(x_ref, win0_ref, wout0_ref, win1_ref, wout1_ref, win2_ref,
             wout2_ref, out_ref, X_ref, P_ref, rs_ref, ssem, rsem):
        my = lax.axis_index("i")
        left = (my - 1) % N_DEV
        right = (my + 1) % N_DEV

        bsem = pltpu.get_barrier_semaphore()

        def barrier():
            for nbr in (left, right):
                pl.semaphore_signal(
                    bsem, inc=1, device_id=(nbr,),
                    device_id_type=pl.DeviceIdType.MESH)
            pl.semaphore_wait(bsem, 2)

        def rows(idx):
            return pl.ds(idx * B_PER, B_PER)

        def ag_phase():
            barrier()
            for h in range(N_DEV - 1):
                o_send = (my - h) % N_DEV
                o_recv = (my - h - 1) % N_DEV
                send = pltpu.make_async_remote_copy(
                    src_ref=X_ref.at[rows(o_send)],
                    dst_ref=X_ref.at[rows(o_send)],
                    send_sem=ssem.at[h], recv_sem=rsem.at[h],
                    device_id=(right,),
                    device_id_type=pl.DeviceIdType.MESH)
                send.start()
                recv = pltpu.make_async_remote_copy(
                    src_ref=X_ref.at[rows(o_recv)],
                    dst_ref=X_ref.at[rows(o_recv)],
                    send_sem=ssem.at[h], recv_sem=rsem.at[h],
                    device_id=(left,),
                    device_id_type=pl.DeviceIdType.MESH)
                recv.wait_recv()
                send.wait_send()

        def rs_phase():
            barrier()
            for s in range(N_DEV - 1):
                c_send = (my - 1 - s) % N_DEV
                if s == 0:
                    src = P_ref.at[rows(c_send)]
                else:
                    src = rs_ref.at[s - 1]
                send = pltpu.make_async_remote_copy(
                    src_ref=src,
                    dst_ref=rs_ref.at[s],
                    send_sem=ssem.at[s], recv_sem=rsem.at[s],
                    device_id=(right,),
                    device_id_type=pl.DeviceIdType.MESH)
                send.start()
                recv = pltpu.make_async_remote_copy(
                    src_ref=rs_ref.at[s],
                    dst_ref=rs_ref.at[s],
                    send_sem=ssem.at[s], recv_sem=rsem.at[s],
                    device_id=(left,),
                    device_id_type=pl.DeviceIdType.MESH)
                recv.wait_recv()
                c_recv = (my - 2 - s) % N_DEV
                rs_ref[s] = rs_ref[s] + P_ref[rows(c_recv), :]
                send.wait_send()

        w_in = (win0_ref, win1_ref, win2_ref)
        w_out = (wout0_ref, wout1_ref, wout2_ref)

        X_ref[rows(my), :] = x_ref[:].astype(BF16)
        ag_phase()

        for L in range(3):
            Xv = X_ref[:]
            Wi = w_in[L][:].astype(BF16)
            Wo = w_out[L][:].astype(BF16)
            Hv = jnp.maximum(
                jnp.dot(Xv, Wi, preferred_element_type=F32), 0.0)
            P = jnp.dot(Hv.astype(BF16), Wo, preferred_element_type=F32)
            P_ref[:] = P
            rs_phase()
            red = rs_ref[N_DEV - 2]
            if L < 2:
                X_ref[rows(my), :] = red.astype(BF16)
                ag_phase()
            else:
                out_ref[:] = red

    return pl.pallas_call(
        body,
        out_shape=jax.ShapeDtypeStruct((B_PER, D), jnp.float32),
        in_specs=[pl.BlockSpec(memory_space=pltpu.VMEM)] * 7,
        out_specs=pl.BlockSpec(memory_space=pltpu.VMEM),
        scratch_shapes=[
            pltpu.VMEM((N_DEV * B_PER, D), BF16),
            pltpu.VMEM((N_DEV * B_PER, D), F32),
            pltpu.VMEM((N_DEV - 1, B_PER, D), F32),
            pltpu.SemaphoreType.DMA((N_DEV - 1,)),
            pltpu.SemaphoreType.DMA((N_DEV - 1,)),
        ],
        compiler_params=pltpu.CompilerParams(collective_id=0),
    )(x, Win0, Wout0, Win1, Wout1, Win2, Wout2)


# device time: 170273 ns/iter; 2.2444x vs baseline; 2.2444x over previous
import jax
import jax.numpy as jnp
from jax import lax
from jax.experimental import pallas as pl
from jax.experimental.pallas import tpu as pltpu

N_DEV = 16
B_PER = 256
D = 256
H_PER = 512
BF16 = jnp.bfloat16
F32 = jnp.float32
MESH = pl.DeviceIdType.MESH


def kernel(x, Win0, Wout0, Win1, Wout1, Win2, Wout2):
    def body(x_ref, win0_ref, wout0_ref, win1_ref, wout1_ref, win2_ref,
             wout2_ref, out_ref, X_ref, P_ref, Pb_ref, rs_ref,
             ssem, rsem_ag, rsem_rs):
        my = lax.axis_index("i")

        def rows(idx):
            return pl.ds(idx * B_PER, B_PER)

        bsem = pltpu.get_barrier_semaphore()
        for k in range(1, N_DEV):
            pl.semaphore_signal(bsem, inc=1, device_id=((my + k) % N_DEV,),
                                device_id_type=MESH)
        pl.semaphore_wait(bsem, N_DEV - 1)

        def ag_phase():
            sends = []
            for k in range(1, N_DEV):
                p = (my + k) % N_DEV
                d = pltpu.make_async_remote_copy(
                    src_ref=X_ref.at[rows(my)],
                    dst_ref=X_ref.at[rows(my)],
                    send_sem=ssem.at[k - 1], recv_sem=rsem_ag.at[k - 1],
                    device_id=(p,), device_id_type=MESH)
                d.start()
                sends.append(d)
            for k in range(1, N_DEV):
                q = (my - k) % N_DEV
                recv = pltpu.make_async_remote_copy(
                    src_ref=X_ref.at[rows(q)],
                    dst_ref=X_ref.at[rows(q)],
                    send_sem=ssem.at[k - 1], recv_sem=rsem_ag.at[k - 1],
                    device_id=(q,), device_id_type=MESH)
                recv.wait_recv()
            for d in sends:
                d.wait_send()

        def rs_phase():
            sends = []
            for k in range(1, N_DEV):
                p = (my + k) % N_DEV
                d = pltpu.make_async_remote_copy(
                    src_ref=Pb_ref.at[rows(p)],
                    dst_ref=rs_ref.at[k - 1],
                    send_sem=ssem.at[k - 1], recv_sem=rsem_rs.at[k - 1],
                    device_id=(p,), device_id_type=MESH)
                d.start()
                sends.append(d)
            for k in range(1, N_DEV):
                q = (my - k) % N_DEV
                recv = pltpu.make_async_remote_copy(
                    src_ref=rs_ref.at[k - 1],
                    dst_ref=rs_ref.at[k - 1],
                    send_sem=ssem.at[k - 1], recv_sem=rsem_rs.at[k - 1],
                    device_id=(q,), device_id_type=MESH)
                recv.wait_recv()
            for d in sends:
                d.wait_send()
            acc = P_ref[rows(my), :]
            for k in range(1, N_DEV):
                acc = acc + rs_ref[k - 1].astype(F32)
            return acc

        w_in = (win0_ref, win1_ref, win2_ref)
        w_out = (wout0_ref, wout1_ref, wout2_ref)

        X_ref[rows(my), :] = x_ref[:].astype(BF16)
        ag_phase()

        for L in range(3):
            Xv = X_ref[:]
            Wi = w_in[L][:].astype(BF16)
            Wo = w_out[L][:].astype(BF16)
            Hv = jnp.maximum(
                jnp.dot(Xv, Wi, preferred_element_type=F32), 0.0)
            P = jnp.dot(Hv.astype(BF16), Wo, preferred_element_type=F32)
            P_ref[:] = P
            Pb_ref[:] = P.astype(BF16)
            red = rs_phase()
            if L < 2:
                X_ref[rows(my), :] = red.astype(BF16)
                ag_phase()
            else:
                out_ref[:] = red

    return pl.pallas_call(
        body,
        out_shape=jax.ShapeDtypeStruct((B_PER, D), jnp.float32),
        in_specs=[pl.BlockSpec(memory_space=pltpu.VMEM)] * 7,
        out_specs=pl.BlockSpec(memory_space=pltpu.VMEM),
        scratch_shapes=[
            pltpu.VMEM((N_DEV * B_PER, D), BF16),
            pltpu.VMEM((N_DEV * B_PER, D), F32),
            pltpu.VMEM((N_DEV * B_PER, D), BF16),
            pltpu.VMEM((N_DEV - 1, B_PER, D), BF16),
            pltpu.SemaphoreType.DMA((N_DEV - 1,)),
            pltpu.SemaphoreType.DMA((N_DEV - 1,)),
            pltpu.SemaphoreType.DMA((N_DEV - 1,)),
        ],
        compiler_params=pltpu.CompilerParams(collective_id=0),
    )(x, Win0, Wout0, Win1, Wout1, Win2, Wout2)


# device time: 165195 ns/iter; 2.3134x vs baseline; 1.0307x over previous
import jax
import jax.numpy as jnp
from jax import lax
from jax.experimental import pallas as pl
from jax.experimental.pallas import tpu as pltpu

N_DEV = 16
B_PER = 256
D = 256
H_PER = 512
BF16 = jnp.bfloat16
F32 = jnp.float32
MESH = pl.DeviceIdType.MESH


def kernel(x, Win0, Wout0, Win1, Wout1, Win2, Wout2):
    def body(x_ref, win0_ref, wout0_ref, win1_ref, wout1_ref, win2_ref,
             wout2_ref, out_ref, X_ref, Pb_ref, rs_ref,
             ssem_ag, ssem_rs, rsem_ag, rsem_rs):
        my = lax.axis_index("i")

        def rows(idx):
            return pl.ds(idx * B_PER, B_PER)

        bsem = pltpu.get_barrier_semaphore()
        for k in range(1, N_DEV):
            pl.semaphore_signal(bsem, inc=1, device_id=((my + k) % N_DEV,),
                                device_id_type=MESH)
        pl.semaphore_wait(bsem, N_DEV - 1)

        def broadcast_own_chunk():
            sends = []
            for k in range(1, N_DEV):
                d = pltpu.make_async_remote_copy(
                    src_ref=X_ref.at[rows(my)],
                    dst_ref=X_ref.at[rows(my)],
                    send_sem=ssem_ag.at[k - 1], recv_sem=rsem_ag.at[k - 1],
                    device_id=((my + k) % N_DEV,), device_id_type=MESH)
                d.start()
                sends.append(d)
            return sends

        w_in = (win0_ref, win1_ref, win2_ref)
        w_out = (wout0_ref, wout1_ref, wout2_ref)

        X_ref[rows(my), :] = x_ref[:].astype(BF16)
        ag_sends = broadcast_own_chunk()

        for L in range(3):
            Wi = w_in[L][:].astype(BF16)
            Wo = w_out[L][:].astype(BF16)
            rs_sends = []
            p_own = None
            for k in range(N_DEV):
                c = (my - k) % N_DEV
                if k > 0:
                    recv = pltpu.make_async_remote_copy(
                        src_ref=X_ref.at[rows(c)],
                        dst_ref=X_ref.at[rows(c)],
                        send_sem=ssem_ag.at[k - 1],
                        recv_sem=rsem_ag.at[k - 1],
                        device_id=(c,), device_id_type=MESH)
                    recv.wait_recv()
                Xc = X_ref[rows(c), :]
                Hc = jnp.maximum(
                    jnp.dot(Xc, Wi, preferred_element_type=F32), 0.0)
                Pc = jnp.dot(Hc.astype(BF16), Wo,
                             preferred_element_type=F32)
                if k == 0:
                    p_own = Pc
                else:
                    Pb_ref[rows(c), :] = Pc.astype(BF16)
                    d = pltpu.make_async_remote_copy(
                        src_ref=Pb_ref.at[rows(c)],
                        dst_ref=rs_ref.at[15 - k],
                        send_sem=ssem_rs.at[15 - k],
                        recv_sem=rsem_rs.at[15 - k],
                        device_id=(c,), device_id_type=MESH)
                    d.start()
                    rs_sends.append(d)

            for d in ag_sends:
                d.wait_send()

            acc = p_own
            for k in range(1, N_DEV):
                q = (my + k) % N_DEV
                recv = pltpu.make_async_remote_copy(
                    src_ref=rs_ref.at[15 - k],
                    dst_ref=rs_ref.at[15 - k],
                    send_sem=ssem_rs.at[15 - k],
                    recv_sem=rsem_rs.at[15 - k],
                    device_id=(q,), device_id_type=MESH)
                recv.wait_recv()
                acc = acc + rs_ref[15 - k].astype(F32)

            if L < 2:
                X_ref[rows(my), :] = acc.astype(BF16)
                ag_sends = broadcast_own_chunk()
            else:
                out_ref[:] = acc
            for d in rs_sends:
                d.wait_send()

    return pl.pallas_call(
        body,
        out_shape=jax.ShapeDtypeStruct((B_PER, D), jnp.float32),
        in_specs=[pl.BlockSpec(memory_space=pltpu.VMEM)] * 7,
        out_specs=pl.BlockSpec(memory_space=pltpu.VMEM),
        scratch_shapes=[
            pltpu.VMEM((N_DEV * B_PER, D), BF16),
            pltpu.VMEM((N_DEV * B_PER, D), BF16),
            pltpu.VMEM((N_DEV - 1, B_PER, D), BF16),
            pltpu.SemaphoreType.DMA((N_DEV - 1,)),
            pltpu.SemaphoreType.DMA((N_DEV - 1,)),
            pltpu.SemaphoreType.DMA((N_DEV - 1,)),
            pltpu.SemaphoreType.DMA((N_DEV - 1,)),
        ],
        compiler_params=pltpu.CompilerParams(collective_id=0),
    )(x, Win0, Wout0, Win1, Wout1, Win2, Wout2)
